# baseline (speedup 1.0000x reference)
import functools

import jax
import jax.numpy as jnp
from jax import lax
from jax.experimental import pallas as pl
from jax.experimental.pallas import tpu as pltpu
from jax.experimental.pallas import tpu_sc as plsc

NC = 2
NS = 16
NW = NC * NS
DP = 128
CH = 128
NBUF = 4
S = 2


@functools.lru_cache(maxsize=None)
def _build_gather(B, V):
    assert B % (NW * CH) == 0
    b_per_w = B // NW
    steps = b_per_w // CH
    assert steps % NBUF == 0 and steps > NBUF and 0 < S < NBUF
    mesh = plsc.VectorSubcoreMesh(core_axis_name="c", subcore_axis_name="s")

    @functools.partial(
        pl.kernel,
        out_type=jax.ShapeDtypeStruct((B, DP), jnp.float32),
        mesh=mesh,
        scratch_types=[
            pltpu.VMEM((b_per_w,), jnp.int32),
            pltpu.VMEM((NBUF, CH, DP), jnp.float32),
            pltpu.SemaphoreType.DMA,
            pltpu.SemaphoreType.DMA,
        ],
        compiler_params=pltpu.CompilerParams(use_tc_tiling_on_sc=True),
    )
    def gather_kernel(idx_hbm, tab_hbm, out_hbm, idx_v, rows_v, sem_g, sem_o):
        wid = lax.axis_index("s") * NC + lax.axis_index("c")
        base = wid * b_per_w
        pltpu.sync_copy(idx_hbm.at[pl.ds(base, b_per_w)], idx_v)

        def start_gather(g, b):
            for j in range(CH // 16):
                iv = idx_v[pl.ds(g * CH + j * 16, 16)]
                pltpu.async_copy(
                    tab_hbm.at[iv], rows_v.at[b].at[pl.ds(j * 16, 16)], sem_g
                )

        def wait_gather(b):
            pltpu.make_async_copy(tab_hbm.at[pl.ds(0, CH)], rows_v.at[b], sem_g).wait()

        def start_wb(g, b):
            pltpu.async_copy(rows_v.at[b], out_hbm.at[pl.ds(base + g * CH, CH)], sem_o)

        def wait_wb(b):
            pltpu.make_async_copy(rows_v.at[b], out_hbm.at[pl.ds(base, CH)], sem_o).wait()

        for b in range(NBUF - S):
            start_gather(b, b)
        for g in range(S):
            start_gather(g + NBUF - S, (g + NBUF - S) % NBUF)
            wait_gather(g % NBUF)
            start_wb(g, g % NBUF)

        @pl.loop(0, steps - NBUF, step=NBUF)
        def _(g0):
            for j in range(NBUF):
                g = g0 + S + j
                wait_wb(j)
                start_gather(g + NBUF - S, j)
                wait_gather((j + S) % NBUF)
                start_wb(g, (j + S) % NBUF)

        for g in range(steps - NBUF + S, steps):
            wait_wb((g - S) % NBUF)
            wait_gather(g % NBUF)
            start_wb(g, g % NBUF)
        for g in range(steps - S, steps):
            wait_wb(g % NBUF)

    return gather_kernel


def kernel(x, lut):
    bt, h = x.shape
    v, d = lut.shape
    b = bt * h
    lut_padded = jnp.pad(lut, ((0, 0), (0, DP - d)))
    out = _build_gather(b, v)(x.reshape(b), lut_padded)
    return out[:, :d].reshape(bt, h, d)


# speedup vs baseline: 1.0018x; 1.0018x over previous
import functools

import jax
import jax.numpy as jnp
from jax import lax
from jax.experimental import pallas as pl
from jax.experimental.pallas import tpu as pltpu
from jax.experimental.pallas import tpu_sc as plsc

NC = 2
NS = 16
NW = NC * NS
DP = 128
CH = 128
NBUF = 5
S = 2


@functools.lru_cache(maxsize=None)
def _build_gather(B, V):
    assert B % (NW * CH) == 0
    b_per_w = B // NW
    steps = b_per_w // CH
    assert steps % NBUF == 0 and steps > NBUF and 0 < S < NBUF
    mesh = plsc.VectorSubcoreMesh(core_axis_name="c", subcore_axis_name="s")

    @functools.partial(
        pl.kernel,
        out_type=jax.ShapeDtypeStruct((B, DP), jnp.float32),
        mesh=mesh,
        scratch_types=[
            pltpu.VMEM((b_per_w,), jnp.int32),
            pltpu.VMEM((NBUF, CH, DP), jnp.float32),
            pltpu.SemaphoreType.DMA,
            pltpu.SemaphoreType.DMA,
        ],
        compiler_params=pltpu.CompilerParams(use_tc_tiling_on_sc=True),
    )
    def gather_kernel(idx_hbm, tab_hbm, out_hbm, idx_v, rows_v, sem_g, sem_o):
        wid = lax.axis_index("s") * NC + lax.axis_index("c")
        base = wid * b_per_w
        pltpu.sync_copy(idx_hbm.at[pl.ds(base, b_per_w)], idx_v)

        def start_gather(g, b):
            for j in range(CH // 16):
                iv = idx_v[pl.ds(g * CH + j * 16, 16)]
                pltpu.async_copy(
                    tab_hbm.at[iv], rows_v.at[b].at[pl.ds(j * 16, 16)], sem_g
                )

        def wait_gather(b):
            pltpu.make_async_copy(tab_hbm.at[pl.ds(0, CH)], rows_v.at[b], sem_g).wait()

        def start_wb(g, b):
            pltpu.async_copy(rows_v.at[b], out_hbm.at[pl.ds(base + g * CH, CH)], sem_o)

        def wait_wb(b):
            pltpu.make_async_copy(rows_v.at[b], out_hbm.at[pl.ds(base, CH)], sem_o).wait()

        for b in range(NBUF - S):
            start_gather(b, b)
        for g in range(S):
            start_gather(g + NBUF - S, (g + NBUF - S) % NBUF)
            wait_gather(g % NBUF)
            start_wb(g, g % NBUF)

        @pl.loop(0, steps - NBUF, step=NBUF)
        def _(g0):
            for j in range(NBUF):
                g = g0 + S + j
                wait_wb(j)
                start_gather(g + NBUF - S, j)
                wait_gather((j + S) % NBUF)
                start_wb(g, (j + S) % NBUF)

        for g in range(steps - NBUF + S, steps):
            wait_wb((g - S) % NBUF)
            wait_gather(g % NBUF)
            start_wb(g, g % NBUF)
        for g in range(steps - S, steps):
            wait_wb(g % NBUF)

    return gather_kernel


def kernel(x, lut):
    bt, h = x.shape
    v, d = lut.shape
    b = bt * h
    lut_padded = jnp.pad(lut, ((0, 0), (0, DP - d)))
    out = _build_gather(b, v)(x.reshape(b), lut_padded)
    return out[:, :d].reshape(bt, h, d)
